# baseline (speedup 1.0000x reference)
import functools

import jax
import jax.numpy as jnp
from jax import lax
from jax.experimental import pallas as pl
from jax.experimental.pallas import tpu as pltpu
from jax.experimental.pallas import tpu_sc as plsc

_MOMENTUM = 0.9
_BLOCK_ROWS = 4000
_NUM_CORES = 2
_NUM_SUBCORES = 16
_LANES = 16


def _copy_body(mem_ref, out_ref):
    out_ref[...] = mem_ref[...]


def _copy_mem(mem):
    num_entries, feat = mem.shape
    return pl.pallas_call(
        _copy_body,
        grid=(num_entries // _BLOCK_ROWS,),
        in_specs=[pl.BlockSpec((_BLOCK_ROWS, feat), lambda i: (i, 0))],
        out_specs=pl.BlockSpec((_BLOCK_ROWS, feat), lambda i: (i, 0)),
        out_shape=jax.ShapeDtypeStruct((num_entries, feat), jnp.float32),
    )(mem)


def _make_sc_update(batch, feat):
    num_workers = _NUM_CORES * _NUM_SUBCORES
    rows_per_worker = batch // num_workers
    chunks = feat // _LANES
    mesh = plsc.VectorSubcoreMesh(core_axis_name="c", subcore_axis_name="s")

    @functools.partial(
        pl.kernel,
        out_type=(),
        mesh=mesh,
        scratch_types=[
            pltpu.VMEM((rows_per_worker,), jnp.int32),
            pltpu.VMEM((rows_per_worker, feat), jnp.float32),
            pltpu.VMEM((rows_per_worker, feat), jnp.float32),
            pltpu.SemaphoreType.DMA,
            pltpu.SemaphoreType.DMA,
        ],
    )
    def _sc_update(vals_hbm, tgt_hbm, mem_hbm, newmem_hbm,
                   idx_v, old_v, val_v, gsem, ssem):
        wid = lax.axis_index("s") * _NUM_CORES + lax.axis_index("c")
        base = wid * rows_per_worker
        pltpu.sync_copy(tgt_hbm.at[pl.ds(base, rows_per_worker)], idx_v)
        gather = pltpu.async_copy(mem_hbm.at[idx_v], old_v, gsem)
        pltpu.sync_copy(vals_hbm.at[pl.ds(base, rows_per_worker)], val_v)
        gather.wait()

        def row(r, carry):
            for c in range(chunks):
                sl = pl.ds(c * _LANES, _LANES)
                old_v[r, sl] = (
                    _MOMENTUM * old_v[r, sl]
                    + (1.0 - _MOMENTUM) * val_v[r, sl]
                )
            return carry

        lax.fori_loop(0, rows_per_worker, row, 0)
        pltpu.async_copy(old_v, newmem_hbm.at[idx_v], ssem).wait()

    return _sc_update


_X_DMA_CHUNKS = 16


def _x_dma_body(x_ref, out_ref, sem):
    rows = x_ref.shape[0] // _X_DMA_CHUNKS
    copies = [
        pltpu.make_async_copy(
            x_ref.at[pl.ds(i * rows, rows)],
            out_ref.at[pl.ds(i * rows, rows)],
            sem,
        )
        for i in range(_X_DMA_CHUNKS)
    ]
    for c in copies:
        c.start()
    for c in copies:
        c.wait()


def _copy_x(x):
    return pl.pallas_call(
        _x_dma_body,
        in_specs=[pl.BlockSpec(memory_space=pltpu.MemorySpace.HBM)],
        out_specs=pl.BlockSpec(memory_space=pltpu.MemorySpace.HBM),
        scratch_shapes=[pltpu.SemaphoreType.DMA],
        out_shape=jax.ShapeDtypeStruct(x.shape, x.dtype),
    )(x)


def kernel(x, targets, mem):
    batch, _, feat = x.shape
    vals = x[:, 0]
    new_mem_ref = jax.new_ref(_copy_mem(mem))
    _make_sc_update(batch, feat)(vals, targets, mem, new_mem_ref)
    return _copy_x(x), new_mem_ref[...]


# speedup vs baseline: 25.2084x; 25.2084x over previous
import functools

import jax
import jax.numpy as jnp
from jax import lax
from jax.experimental import pallas as pl
from jax.experimental.pallas import tpu as pltpu
from jax.experimental.pallas import tpu_sc as plsc

_MOMENTUM = 0.9
_BLOCK_ROWS = 4000
_NUM_CORES = 2
_NUM_SUBCORES = 16
_LANES = 16


def _copy_body(mem_ref, out_ref):
    out_ref[...] = mem_ref[...]


def _copy_mem(mem):
    num_entries, feat = mem.shape
    return pl.pallas_call(
        _copy_body,
        grid=(num_entries // _BLOCK_ROWS,),
        in_specs=[pl.BlockSpec((_BLOCK_ROWS, feat), lambda i: (i, 0))],
        out_specs=pl.BlockSpec((_BLOCK_ROWS, feat), lambda i: (i, 0)),
        out_shape=jax.ShapeDtypeStruct((num_entries, feat), jnp.float32),
    )(mem)


def _make_sc_update(batch, feat):
    num_workers = _NUM_CORES * _NUM_SUBCORES
    rows_per_worker = batch // num_workers
    chunks = feat // _LANES
    mesh = plsc.VectorSubcoreMesh(core_axis_name="c", subcore_axis_name="s")

    @functools.partial(
        pl.kernel,
        out_type=(),
        mesh=mesh,
        scratch_types=[
            pltpu.VMEM((rows_per_worker,), jnp.int32),
            pltpu.VMEM((rows_per_worker, feat), jnp.float32),
            pltpu.VMEM((rows_per_worker, feat), jnp.float32),
            pltpu.SemaphoreType.DMA,
            pltpu.SemaphoreType.DMA,
        ],
    )
    def _sc_update(vals_hbm, tgt_hbm, mem_hbm, newmem_hbm,
                   idx_v, old_v, val_v, gsem, ssem):
        wid = lax.axis_index("s") * _NUM_CORES + lax.axis_index("c")
        base = wid * rows_per_worker
        pltpu.sync_copy(tgt_hbm.at[pl.ds(base, rows_per_worker)], idx_v)
        gather = pltpu.async_copy(mem_hbm.at[idx_v], old_v, gsem)
        pltpu.sync_copy(vals_hbm.at[pl.ds(base, rows_per_worker)], val_v)
        gather.wait()

        def row(r, carry):
            for c in range(chunks):
                sl = pl.ds(c * _LANES, _LANES)
                old_v[r, sl] = (
                    _MOMENTUM * old_v[r, sl]
                    + (1.0 - _MOMENTUM) * val_v[r, sl]
                )
            return carry

        lax.fori_loop(0, rows_per_worker, row, 0)
        pltpu.async_copy(old_v, newmem_hbm.at[idx_v], ssem).wait()

    return _sc_update


_X_DMA_CHUNKS = 16


def _x_dma_body(x_ref, out_ref, sem):
    rows = x_ref.shape[0] // _X_DMA_CHUNKS
    copies = [
        pltpu.make_async_copy(
            x_ref.at[pl.ds(i * rows, rows)],
            out_ref.at[pl.ds(i * rows, rows)],
            sem,
        )
        for i in range(_X_DMA_CHUNKS)
    ]
    for c in copies:
        c.start()
    for c in copies:
        c.wait()


def _copy_x(x):
    return pl.pallas_call(
        _x_dma_body,
        in_specs=[pl.BlockSpec(memory_space=pltpu.MemorySpace.HBM)],
        out_specs=pl.BlockSpec(memory_space=pltpu.MemorySpace.HBM),
        scratch_shapes=[pltpu.SemaphoreType.DMA],
        out_shape=jax.ShapeDtypeStruct(x.shape, x.dtype),
    )(x)


def kernel(x, targets, mem):
    batch, _, feat = x.shape
    vals = x[:, 0]
    new_mem_ref = jax.new_ref(mem)
    _make_sc_update(batch, feat)(vals, targets, mem, new_mem_ref)
    return x, new_mem_ref[...]
